# baseline (speedup 1.0000x reference)
import functools

import jax
import jax.numpy as jnp
from jax import lax
from jax.experimental import pallas as pl
from jax.experimental.pallas import tpu as pltpu
from jax.experimental.pallas import tpu_sc as plsc

N_NODES = 10000
N_EDGES = 320000
D = 128

NC = 2
NS = 16
DH = D // NC

CSZ = 128
CH = 160
IB = 8
NBLK = CH // IB
NBUF = 2
EPT = CH * CSZ
E_PAD = NS * EPT

N_PAD = 10240
RPT = N_PAD // NS
RCH = RPT // CSZ
PAD_DST = N_NODES + 8

_sc_mesh = plsc.VectorSubcoreMesh(core_axis_name="c", subcore_axis_name="s")


@functools.partial(
    pl.kernel,
    out_type=jax.ShapeDtypeStruct((NC, N_PAD, DH), jnp.float32),
    mesh=_sc_mesh,
    scratch_types=[
        pltpu.VMEM_SHARED((N_PAD, DH), jnp.float32),
        pltpu.VMEM_SHARED((N_PAD, DH), jnp.float32),
        [pltpu.VMEM((2 * IB, CSZ), jnp.int32)] * 2,
        [pltpu.VMEM((CSZ, DH), jnp.float32)] * NBUF,
        [pltpu.SemaphoreType.DMA] * NBUF,
        [pltpu.SemaphoreType.DMA] * NBUF,
        [pltpu.SemaphoreType.DMA] * 2,
    ],
)
def _sc_aggregate(x_hbm, idx_hbm, out_hbm,
                  tbl, acc, idxs, bufs, gsems, ssems, isems):
    cid = lax.axis_index("c")
    sid = lax.axis_index("s")
    r0 = sid * RPT

    rows = pl.ds(r0, RPT)
    pltpu.sync_copy(x_hbm.at[cid, rows], tbl.at[rows])
    pltpu.sync_copy(x_hbm.at[cid, rows], acc.at[rows])
    plsc.subcore_barrier()

    def gath(iv, j, s):
        pltpu.async_copy(tbl.at[iv.at[j]], bufs[s], gsems[s])

    def wait_gath(iv, s):
        pltpu.make_async_copy(tbl.at[iv.at[0]], bufs[s], gsems[s]).wait()

    def scat(iv, j, s):
        pltpu.async_copy(bufs[s], acc.at[iv.at[IB + j]], ssems[s], add=True)

    def wait_scat(iv, s):
        pltpu.make_async_copy(bufs[s], acc.at[iv.at[IB]], ssems[s]).wait()

    def fetch_idx(blk, which):
        pltpu.async_copy(idx_hbm.at[sid, blk], idxs[which], isems[which])

    def wait_idx(which):
        pltpu.make_async_copy(idx_hbm.at[sid, 0], idxs[which],
                              isems[which]).wait()

    def process_block(cur, nxt, nxt_ready, chain_wait):
        for p in range(IB // 2):
            a = 2 * p
            b = a + 1
            wait_gath(cur, 0)
            scat(cur, a, 0)
            wait_gath(cur, 1)
            scat(cur, b, 1)
            if p < IB // 2 - 1:
                wait_scat(cur, 0)
                gath(cur, a + 2, 0)
                wait_scat(cur, 1)
                gath(cur, b + 2, 1)
            else:
                @pl.when(nxt_ready)
                def _chain():
                    chain_wait()
                    wait_scat(cur, 0)
                    gath(nxt, 0, 0)
                    wait_scat(cur, 1)
                    gath(nxt, 1, 1)

                @pl.when(jnp.logical_not(nxt_ready))
                def _drain():
                    wait_scat(cur, 0)
                    wait_scat(cur, 1)

    pltpu.sync_copy(idx_hbm.at[sid, 0], idxs[0])
    fetch_idx(1, 1)
    gath(idxs[0], 0, 0)
    gath(idxs[0], 1, 1)

    def body(i, carry):
        blk = 2 * i
        process_block(idxs[0], idxs[1], jnp.bool_(True),
                      lambda: wait_idx(1))
        @pl.when(blk + 2 < NBLK)
        def _pf0():
            fetch_idx(blk + 2, 0)
        process_block(idxs[1], idxs[0], blk + 2 < NBLK,
                      lambda: wait_idx(0))
        @pl.when(blk + 3 < NBLK)
        def _pf1():
            fetch_idx(blk + 3, 1)
        return carry

    lax.fori_loop(0, NBLK // 2, body, 0)
    plsc.subcore_barrier()

    pltpu.sync_copy(acc.at[rows], out_hbm.at[cid, rows])


_ROWS_BLK = 10240


def _mlp_body(split_out, plo_ref, phi_ref, wa_ref, ba_ref, wb_ref,
              bb_ref, o_ref):
    h = jnp.concatenate([plo_ref[...], phi_ref[...]], axis=-1)
    h = jnp.dot(h, wa_ref[...], preferred_element_type=jnp.float32)
    h = jnp.maximum(h + ba_ref[...], 0.0)
    o = jnp.dot(h, wb_ref[...], preferred_element_type=jnp.float32)
    o = o + bb_ref[...]
    if split_out:
        o = jnp.maximum(o, 0.0)
        o_ref[0] = o[:, :DH]
        o_ref[1] = o[:, DH:]
    else:
        o_ref[...] = o


def _mlp(plo, phi, wa, ba, wb, bb, split_out):
    half_spec = pl.BlockSpec((_ROWS_BLK, DH), lambda i: (i, 0))
    full_spec = pl.BlockSpec((D, D), lambda i: (0, 0))
    bias_spec = pl.BlockSpec((1, D), lambda i: (0, 0))
    if split_out:
        out_spec = pl.BlockSpec((NC, _ROWS_BLK, DH), lambda i: (0, i, 0))
        out_shape = jax.ShapeDtypeStruct((NC, N_PAD, DH), jnp.float32)
    else:
        out_spec = pl.BlockSpec((_ROWS_BLK, D), lambda i: (i, 0))
        out_shape = jax.ShapeDtypeStruct((N_PAD, D), jnp.float32)
    return pl.pallas_call(
        functools.partial(_mlp_body, split_out),
        grid=(N_PAD // _ROWS_BLK,),
        in_specs=[half_spec, half_spec,
                  full_spec, bias_spec, full_spec, bias_spec],
        out_specs=out_spec,
        out_shape=out_shape,
    )(plo, phi, wa, ba.reshape(1, D), wb, bb.reshape(1, D))


def kernel(x, edge_index, W1a, b1a, W1b, b1b, W2a, b2a, W2b, b2b):
    src = edge_index[0].astype(jnp.int32)
    dst = edge_index[1].astype(jnp.int32)
    pad_e = E_PAD - N_EDGES
    src_r = jnp.concatenate([src, jnp.zeros((pad_e,), jnp.int32)])
    src_r = src_r.reshape(NS, NBLK, IB, CSZ)
    dst_r = jnp.concatenate([dst, jnp.full((pad_e,), PAD_DST, jnp.int32)])
    dst_r = dst_r.reshape(NS, NBLK, IB, CSZ)
    idx_comb = jnp.concatenate([src_r, dst_r], axis=2)
    x_pad = jnp.concatenate(
        [x, jnp.zeros((N_PAD - N_NODES, D), jnp.float32)])
    x2 = jnp.stack([x_pad[:, :DH], x_pad[:, DH:]])

    parts1 = _sc_aggregate(x2, idx_comb)
    h1_2 = _mlp(parts1[0], parts1[1], W1a, b1a, W1b, b1b, split_out=True)
    parts2 = _sc_aggregate(h1_2, idx_comb)
    out = _mlp(parts2[0], parts2[1], W2a, b2a, W2b, b2b, split_out=False)
    return out[:N_NODES]
